# baseline (speedup 1.0000x reference)
import functools

import jax
import jax.numpy as jnp
from jax import lax
from jax.experimental import pallas as pl
from jax.experimental.pallas import tpu as pltpu
from jax.experimental.pallas import tpu_sc as plsc

CHUNK = 128
GROUP = 8
VCHUNK = 12800
LANES = 128


def _tc_repack(t_dv, g, v, d, nf):
    fold = LANES // d
    start = g * fold
    nv = (v + VCHUNK - 1) // VCHUNK

    def body(x_ref, o_ref):
        acc = None
        for k in range(nf):
            e_k = (
                lax.broadcasted_iota(jnp.int32, (d, LANES), 0) + k * d
                == lax.broadcasted_iota(jnp.int32, (d, LANES), 1)
            ).astype(jnp.float32)
            p = lax.dot_general(
                x_ref[k],
                e_k,
                (((0,), (0,)), ((), ())),
                preferred_element_type=jnp.float32,
            )
            acc = p if acc is None else acc + p
        o_ref[...] = acc

    return pl.pallas_call(
        body,
        grid=(nv,),
        in_specs=[pl.BlockSpec((nf, d, VCHUNK), lambda j: (start // nf, 0, j))],
        out_specs=pl.BlockSpec((VCHUNK, LANES), lambda j: (j, 0)),
        out_shape=jax.ShapeDtypeStruct((v, LANES), jnp.float32),
    )(t_dv)


def _sc_gather(flat_idx, flat_tables, g, nw, rpw, d):
    nch = rpw // CHUNK
    ngroups = nch // GROUP
    grows = GROUP * CHUNK
    mesh = plsc.VectorSubcoreMesh(core_axis_name="c", subcore_axis_name="s")
    nc = 2

    @functools.partial(
        pl.kernel,
        mesh=mesh,
        compiler_params=pltpu.CompilerParams(use_tc_tiling_on_sc=False),
        out_type=jax.ShapeDtypeStruct((nw, rpw, d), jnp.float32),
        scratch_types=[
            pltpu.VMEM((nch, CHUNK), jnp.int32),
            pltpu.VMEM((grows, d), jnp.float32),
            pltpu.SemaphoreType.DMA,
        ],
    )
    def k(idx_hbm, tab_hbm, out_hbm, idx_v, rows_v, sem):
        wid = lax.axis_index("s") * nc + lax.axis_index("c")
        pltpu.sync_copy(idx_hbm.at[g, wid], idx_v)

        def body(gg, carry):
            handles = []
            for j in range(GROUP):
                h = pltpu.async_copy(
                    tab_hbm.at[idx_v.at[gg * GROUP + j]],
                    rows_v.at[pl.ds(j * CHUNK, CHUNK)],
                    sem,
                )
                handles.append(h)
            for h in handles:
                h.wait()
            pltpu.sync_copy(rows_v, out_hbm.at[wid, pl.ds(gg * grows, grows)])
            return carry

        lax.fori_loop(0, ngroups, body, 0)

    return k(flat_idx, flat_tables)


def _tc_mlp(xs, w1s, b1, W2, b2, tb=1024):
    bsz = xs[0].shape[0]
    h1 = w1s[0].shape[1]
    ed = W2.shape[1]
    ng = len(xs)

    def body(*refs):
        x_refs = refs[:ng]
        w_refs = refs[ng:2 * ng]
        b1_ref, w2_ref, b2_ref, o_ref = refs[2 * ng:]
        h = b1_ref[...]
        for xr, wr in zip(x_refs, w_refs):
            h = h + jnp.dot(xr[...], wr[...], preferred_element_type=jnp.float32)
        h = jnp.maximum(h, 0.0)
        o = jnp.dot(h, w2_ref[...], preferred_element_type=jnp.float32)
        o_ref[...] = jnp.maximum(o + b2_ref[...], 0.0)

    in_specs = (
        [pl.BlockSpec((tb, x.shape[1]), lambda i: (i, 0)) for x in xs]
        + [pl.BlockSpec(w.shape, lambda i: (0, 0)) for w in w1s]
        + [
            pl.BlockSpec((1, h1), lambda i: (0, 0)),
            pl.BlockSpec(W2.shape, lambda i: (0, 0)),
            pl.BlockSpec((1, ed), lambda i: (0, 0)),
        ]
    )
    return pl.pallas_call(
        body,
        grid=(bsz // tb,),
        in_specs=in_specs,
        out_specs=pl.BlockSpec((tb, ed), lambda i: (i, 0)),
        out_shape=jax.ShapeDtypeStruct((bsz, ed), jnp.float32),
    )(*xs, *w1s, b1.reshape(1, h1), W2, b2.reshape(1, ed))


def kernel(indices, tables, W1, b1, W2, b2):
    f, b = indices.shape
    _, v, d = tables.shape
    nw = 32
    fold = LANES // d

    t_dv = jnp.transpose(tables, (0, 2, 1))
    idx32 = indices.astype(jnp.int32)

    ngrp = (f + fold - 1) // fold
    fpad = ngrp * fold
    rpw = b * fold // nw

    idx_pad = jnp.pad(idx32, ((0, fpad - f), (0, 0)))
    flat_all = (
        jnp.transpose(idx_pad.reshape(ngrp, fold, b), (0, 2, 1)) * fold
        + jnp.arange(fold, dtype=jnp.int32)[None, None, :]
    ).reshape(ngrp, nw, rpw // CHUNK, CHUNK)

    w1_pad = jnp.pad(W1, ((0, (fpad - f) * d), (0, 0)))

    xs, w1s = [], []
    for g in range(ngrp):
        nf = min(fold, f - g * fold)
        packed = _tc_repack(t_dv, g, v, d, nf)
        rows = _sc_gather(flat_all, packed.reshape(v * fold, d), g, nw, rpw, d)
        xs.append(rows.reshape(b, fold * d))
        w1s.append(w1_pad[g * fold * d : (g + 1) * fold * d])

    return _tc_mlp(xs, w1s, b1, W2, b2)


# speedup vs baseline: 1.3393x; 1.3393x over previous
import functools

import jax
import jax.numpy as jnp
from jax import lax
from jax.experimental import pallas as pl
from jax.experimental.pallas import tpu as pltpu
from jax.experimental.pallas import tpu_sc as plsc

CHUNK = 128
GROUP = 8
VCHUNK = 12800
LANES = 128


def _tc_repack(t_dv, g, v, d, nf):
    fold = LANES // d
    start = g * fold
    nv = (v + VCHUNK - 1) // VCHUNK

    def body(x_ref, o_ref):
        acc = None
        for k in range(nf):
            e_k = (
                lax.broadcasted_iota(jnp.int32, (d, LANES), 0) + k * d
                == lax.broadcasted_iota(jnp.int32, (d, LANES), 1)
            ).astype(jnp.float32)
            p = lax.dot_general(
                x_ref[k],
                e_k,
                (((0,), (0,)), ((), ())),
                preferred_element_type=jnp.float32,
            )
            acc = p if acc is None else acc + p
        o_ref[...] = acc

    return pl.pallas_call(
        body,
        grid=(nv,),
        in_specs=[pl.BlockSpec((nf, d, VCHUNK), lambda j: (start // nf, 0, j))],
        out_specs=pl.BlockSpec((VCHUNK, LANES), lambda j: (j, 0)),
        out_shape=jax.ShapeDtypeStruct((v, LANES), jnp.float32),
    )(t_dv)


def _sc_gather(idx2d, flat_tables, g, nw, d):
    f, b = idx2d.shape
    fold = LANES // d
    bw = b // nw
    nch = bw // CHUNK
    mesh = plsc.VectorSubcoreMesh(core_axis_name="c", subcore_axis_name="s")
    nc = 2

    @functools.partial(
        pl.kernel,
        mesh=mesh,
        compiler_params=pltpu.CompilerParams(use_tc_tiling_on_sc=False),
        out_type=jax.ShapeDtypeStruct((fold, nw, bw, d), jnp.float32),
        scratch_types=[
            pltpu.VMEM((bw,), jnp.int32),
            pltpu.VMEM((bw, d), jnp.float32),
            pltpu.SemaphoreType.DMA,
        ],
    )
    def k(idx_hbm, tab_hbm, out_hbm, idx_v, rows_v, sem):
        wid = lax.axis_index("s") * nc + lax.axis_index("c")

        for fp in range(fold):
            f_src = min(g * fold + fp, f - 1)
            pltpu.sync_copy(idx_hbm.at[f_src, pl.ds(wid * bw, bw)], idx_v)

            def tbody(c, carry, fp=fp):
                idx_v[pl.ds(c * 16, 16)] = idx_v[pl.ds(c * 16, 16)] * fold + fp
                return carry

            lax.fori_loop(0, bw // 16, tbody, 0)

            handles = []
            for j in range(nch):
                h = pltpu.async_copy(
                    tab_hbm.at[idx_v.at[pl.ds(j * CHUNK, CHUNK)]],
                    rows_v.at[pl.ds(j * CHUNK, CHUNK)],
                    sem,
                )
                handles.append(h)
            for h in handles:
                h.wait()
            pltpu.sync_copy(rows_v, out_hbm.at[fp, wid])

    return k(idx2d, flat_tables)


def _tc_mlp(xs, w1s, b1, W2, b2, tb=1024):
    fold, bsz, d = xs[0].shape
    h1 = w1s[0].shape[1]
    ed = W2.shape[1]
    ng = len(xs)

    def body(*refs):
        x_refs = refs[:ng]
        w_refs = refs[ng:2 * ng]
        b1_ref, w2_ref, b2_ref, o_ref = refs[2 * ng:]
        h = b1_ref[...]
        for xr, wr in zip(x_refs, w_refs):
            for k in range(fold):
                h = h + jnp.dot(
                    xr[k],
                    wr[k * d : (k + 1) * d, :],
                    preferred_element_type=jnp.float32,
                )
        h = jnp.maximum(h, 0.0)
        o = jnp.dot(h, w2_ref[...], preferred_element_type=jnp.float32)
        o_ref[...] = jnp.maximum(o + b2_ref[...], 0.0)

    in_specs = (
        [pl.BlockSpec((fold, tb, d), lambda i: (0, i, 0)) for x in xs]
        + [pl.BlockSpec(w.shape, lambda i: (0, 0)) for w in w1s]
        + [
            pl.BlockSpec((1, h1), lambda i: (0, 0)),
            pl.BlockSpec(W2.shape, lambda i: (0, 0)),
            pl.BlockSpec((1, ed), lambda i: (0, 0)),
        ]
    )
    return pl.pallas_call(
        body,
        grid=(bsz // tb,),
        in_specs=in_specs,
        out_specs=pl.BlockSpec((tb, ed), lambda i: (i, 0)),
        out_shape=jax.ShapeDtypeStruct((bsz, ed), jnp.float32),
    )(*xs, *w1s, b1.reshape(1, h1), W2, b2.reshape(1, ed))


def kernel(indices, tables, W1, b1, W2, b2):
    f, b = indices.shape
    _, v, d = tables.shape
    nw = 32
    fold = LANES // d

    t_dv = jnp.transpose(tables, (0, 2, 1))
    idx32 = indices.astype(jnp.int32)

    ngrp = (f + fold - 1) // fold
    fpad = ngrp * fold
    w1_pad = jnp.pad(W1, ((0, (fpad - f) * d), (0, 0)))

    xs, w1s = [], []
    for g in range(ngrp):
        nf = min(fold, f - g * fold)
        packed = _tc_repack(t_dv, g, v, d, nf)
        rows = _sc_gather(idx32, packed.reshape(v * fold, d), g, nw, d)
        xs.append(rows.reshape(fold, b, d))
        w1s.append(w1_pad[g * fold * d : (g + 1) * fold * d])

    return _tc_mlp(xs, w1s, b1, W2, b2)


# speedup vs baseline: 1.9097x; 1.4259x over previous
import functools

import jax
import jax.numpy as jnp
from jax import lax
from jax.experimental import pallas as pl
from jax.experimental.pallas import tpu as pltpu
from jax.experimental.pallas import tpu_sc as plsc

CHUNK = 128
GROUP = 8
VCHUNK = 12800
LANES = 128


def _tc_repack(t_dv, g, v, d, nf):
    fold = LANES // d
    start = g * fold
    nv = (v + VCHUNK - 1) // VCHUNK

    def body(x_ref, o_ref):
        acc = None
        for k in range(nf):
            e_k = (
                lax.broadcasted_iota(jnp.int32, (d, LANES), 0) + k * d
                == lax.broadcasted_iota(jnp.int32, (d, LANES), 1)
            ).astype(jnp.float32)
            p = lax.dot_general(
                x_ref[k],
                e_k,
                (((0,), (0,)), ((), ())),
                preferred_element_type=jnp.float32,
            )
            acc = p if acc is None else acc + p
        o_ref[...] = acc

    return pl.pallas_call(
        body,
        grid=(nv,),
        in_specs=[pl.BlockSpec((nf, d, VCHUNK), lambda j: (start // nf, 0, j))],
        out_specs=pl.BlockSpec((VCHUNK, LANES), lambda j: (j, 0)),
        out_shape=jax.ShapeDtypeStruct((v, LANES), jnp.float32),
    )(t_dv)


def _sc_gather(idx2d, flat_tables, g, nw, d):
    f, b = idx2d.shape
    fold = LANES // d
    bw = b // nw
    nch = bw // CHUNK
    mesh = plsc.VectorSubcoreMesh(core_axis_name="c", subcore_axis_name="s")
    nc = 2

    @functools.partial(
        pl.kernel,
        mesh=mesh,
        compiler_params=pltpu.CompilerParams(use_tc_tiling_on_sc=False),
        out_type=jax.ShapeDtypeStruct((nw, bw, fold, d), jnp.float32),
    scratch_types=[
            pltpu.VMEM((bw,), jnp.int32),
            pltpu.VMEM((bw, d), jnp.float32),
            pltpu.SemaphoreType.DMA,
        ],
    )
    def k(idx_hbm, tab_hbm, out_hbm, idx_v, slab_v, sem):
        wid = lax.axis_index("s") * nc + lax.axis_index("c")

        for fp in range(fold):
            f_src = min(g * fold + fp, f - 1)
            pltpu.sync_copy(idx_hbm.at[f_src, pl.ds(wid * bw, bw)], idx_v)

            def tbody(c, carry, fp=fp):
                idx_v[pl.ds(c * 16, 16)] = idx_v[pl.ds(c * 16, 16)] * fold + fp
                return carry

            lax.fori_loop(0, bw // 16, tbody, 0)

            handles = []
            for j in range(nch):
                h = pltpu.async_copy(
                    tab_hbm.at[idx_v.at[pl.ds(j * CHUNK, CHUNK)]],
                    slab_v.at[pl.ds(j * CHUNK, CHUNK)],
                    sem,
                )
                handles.append(h)
            for h in handles:
                h.wait()
            pltpu.sync_copy(slab_v, out_hbm.at[wid, :, fp])

    return k(idx2d, flat_tables)


def _tc_mlp(xs, w1s, b1, W2, b2, tb=1024):
    bsz, fd = xs[0].shape
    fold, d = 4, fd // 4
    h1 = w1s[0].shape[1]
    ed = W2.shape[1]
    ng = len(xs)

    def body(*refs):
        x_refs = refs[:ng]
        w_refs = refs[ng:2 * ng]
        b1_ref, w2_ref, b2_ref, o_ref = refs[2 * ng:]
        h = b1_ref[...]
        for xr, wr in zip(x_refs, w_refs):
            h = h + jnp.dot(xr[...], wr[...], preferred_element_type=jnp.float32)
        h = jnp.maximum(h, 0.0)
        o = jnp.dot(h, w2_ref[...], preferred_element_type=jnp.float32)
        o_ref[...] = jnp.maximum(o + b2_ref[...], 0.0)

    in_specs = (
        [pl.BlockSpec((tb, fold * d), lambda i: (i, 0)) for x in xs]
        + [pl.BlockSpec(w.shape, lambda i: (0, 0)) for w in w1s]
        + [
            pl.BlockSpec((1, h1), lambda i: (0, 0)),
            pl.BlockSpec(W2.shape, lambda i: (0, 0)),
            pl.BlockSpec((1, ed), lambda i: (0, 0)),
        ]
    )
    return pl.pallas_call(
        body,
        grid=(bsz // tb,),
        in_specs=in_specs,
        out_specs=pl.BlockSpec((tb, ed), lambda i: (i, 0)),
        out_shape=jax.ShapeDtypeStruct((bsz, ed), jnp.float32),
    )(*xs, *w1s, b1.reshape(1, h1), W2, b2.reshape(1, ed))


def kernel(indices, tables, W1, b1, W2, b2):
    f, b = indices.shape
    _, v, d = tables.shape
    nw = 32
    fold = LANES // d

    t_dv = jnp.transpose(tables, (0, 2, 1))
    idx32 = indices.astype(jnp.int32)

    ngrp = (f + fold - 1) // fold
    fpad = ngrp * fold
    w1_pad = jnp.pad(W1, ((0, (fpad - f) * d), (0, 0)))

    xs, w1s = [], []
    for g in range(ngrp):
        nf = min(fold, f - g * fold)
        packed = _tc_repack(t_dv, g, v, d, nf)
        rows = _sc_gather(idx32, packed.reshape(v * fold, d), g, nw, d)
        xs.append(rows.reshape(b, fold * d))
        w1s.append(w1_pad[g * fold * d : (g + 1) * fold * d])

    return _tc_mlp(xs, w1s, b1, W2, b2)


# speedup vs baseline: 2.5393x; 1.3297x over previous
import functools

import jax
import jax.numpy as jnp
from jax import lax
from jax.experimental import pallas as pl
from jax.experimental.pallas import tpu as pltpu
from jax.experimental.pallas import tpu_sc as plsc

CHUNK = 128
GROUP = 8
VCHUNK = 12800
LANES = 128


def _tc_repack(t_dv, g0, ngc, v, d, nf):
    fold = LANES // d
    nv = (v + VCHUNK - 1) // VCHUNK

    def body(x_ref, o_ref):
        acc = None
        for k in range(nf):
            e_k = jnp.asarray(
                lax.broadcasted_iota(jnp.int32, (d, LANES), 0) + k * d
                == lax.broadcasted_iota(jnp.int32, (d, LANES), 1),
                dtype=jnp.bfloat16,
            )
            p = lax.dot_general(
                x_ref[k].astype(jnp.bfloat16),
                e_k,
                (((0,), (0,)), ((), ())),
                preferred_element_type=jnp.float32,
            )
            acc = p if acc is None else acc + p
        o_ref[0] = acc

    return pl.pallas_call(
        body,
        grid=(ngc, nv),
        in_specs=[
            pl.BlockSpec(
                (nf, d, VCHUNK), lambda gg, j: ((g0 * fold) // nf + gg, 0, j)
            )
        ],
        out_specs=pl.BlockSpec((1, VCHUNK, LANES), lambda gg, j: (gg, j, 0)),
        out_shape=jax.ShapeDtypeStruct((ngc, v, LANES), jnp.float32),
    )(t_dv)


def _sc_gather(idx2d, flat_tables, g, base_row, nw, d):
    f, b = idx2d.shape
    fold = LANES // d
    bw = b // nw
    nch = bw // CHUNK
    mesh = plsc.VectorSubcoreMesh(core_axis_name="c", subcore_axis_name="s")
    nc = 2

    @functools.partial(
        pl.kernel,
        mesh=mesh,
        compiler_params=pltpu.CompilerParams(use_tc_tiling_on_sc=False),
        out_type=jax.ShapeDtypeStruct((nw, bw, fold, d), jnp.float32),
    scratch_types=[
            pltpu.VMEM((bw,), jnp.int32),
            pltpu.VMEM((bw, d), jnp.float32),
            pltpu.SemaphoreType.DMA,
        ],
    )
    def k(idx_hbm, tab_hbm, out_hbm, idx_v, slab_v, sem):
        wid = lax.axis_index("s") * nc + lax.axis_index("c")

        for fp in range(fold):
            f_src = min(g * fold + fp, f - 1)
            pltpu.sync_copy(idx_hbm.at[f_src, pl.ds(wid * bw, bw)], idx_v)

            def tbody(c, carry, fp=fp):
                idx_v[pl.ds(c * 16, 16)] = (
                    idx_v[pl.ds(c * 16, 16)] * fold + (base_row + fp)
                )
                return carry

            lax.fori_loop(0, bw // 16, tbody, 0)

            handles = []
            for j in range(nch):
                h = pltpu.async_copy(
                    tab_hbm.at[idx_v.at[pl.ds(j * CHUNK, CHUNK)]],
                    slab_v.at[pl.ds(j * CHUNK, CHUNK)],
                    sem,
                )
                handles.append(h)
            for h in handles:
                h.wait()
            pltpu.sync_copy(slab_v, out_hbm.at[wid, :, fp])

    return k(idx2d, flat_tables)


def _tc_mlp(xs, w1s, b1, W2, b2, tb=1024):
    bsz, fd = xs[0].shape
    fold, d = 4, fd // 4
    h1 = w1s[0].shape[1]
    ed = W2.shape[1]
    ng = len(xs)

    def body(*refs):
        x_refs = refs[:ng]
        w_refs = refs[ng:2 * ng]
        b1_ref, w2_ref, b2_ref, o_ref = refs[2 * ng:]
        h = b1_ref[...]
        for xr, wr in zip(x_refs, w_refs):
            h = h + jnp.dot(xr[...], wr[...], preferred_element_type=jnp.float32)
        h = jnp.maximum(h, 0.0)
        o = jnp.dot(h, w2_ref[...], preferred_element_type=jnp.float32)
        o_ref[...] = jnp.maximum(o + b2_ref[...], 0.0)

    in_specs = (
        [pl.BlockSpec((tb, fold * d), lambda i: (i, 0)) for x in xs]
        + [pl.BlockSpec(w.shape, lambda i: (0, 0)) for w in w1s]
        + [
            pl.BlockSpec((1, h1), lambda i: (0, 0)),
            pl.BlockSpec(W2.shape, lambda i: (0, 0)),
            pl.BlockSpec((1, ed), lambda i: (0, 0)),
        ]
    )
    return pl.pallas_call(
        body,
        grid=(bsz // tb,),
        in_specs=in_specs,
        out_specs=pl.BlockSpec((tb, ed), lambda i: (i, 0)),
        out_shape=jax.ShapeDtypeStruct((bsz, ed), jnp.float32),
    )(*xs, *w1s, b1.reshape(1, h1), W2, b2.reshape(1, ed))


def kernel(indices, tables, W1, b1, W2, b2):
    f, b = indices.shape
    _, v, d = tables.shape
    nw = 32
    fold = LANES // d

    t_dv = jnp.transpose(tables, (0, 2, 1))
    idx32 = indices.astype(jnp.int32)

    ngrp = (f + fold - 1) // fold
    fpad = ngrp * fold
    w1_pad = jnp.pad(W1, ((0, (fpad - f) * d), (0, 0)))

    calls = []
    g = 0
    while g < ngrp:
        nf = min(fold, f - g * fold)
        ngc = 2 if (nf == fold and g + 1 < ngrp and f - (g + 1) * fold >= fold) else 1
        calls.append((g, ngc, nf))
        g += ngc

    xs, w1s = [], []
    for g0, ngc, nf in calls:
        packed = _tc_repack(t_dv, g0, ngc, v, d, nf)
        flat_tab = packed.reshape(ngc * v * fold, d)
        for gg in range(ngc):
            g = g0 + gg
            rows = _sc_gather(idx32, flat_tab, g, gg * v * fold, nw, d)
            xs.append(rows.reshape(b, fold * d))
            w1s.append(w1_pad[g * fold * d : (g + 1) * fold * d])

    return _tc_mlp(xs, w1s, b1, W2, b2)


# speedup vs baseline: 2.5429x; 1.0014x over previous
import functools

import jax
import jax.numpy as jnp
from jax import lax
from jax.experimental import pallas as pl
from jax.experimental.pallas import tpu as pltpu
from jax.experimental.pallas import tpu_sc as plsc

CHUNK = 128
GROUP = 8
VCHUNK = 12800
LANES = 128


def _tc_repack(t_dv, g0, ngc, v, d, nf):
    fold = LANES // d
    nv = (v + VCHUNK - 1) // VCHUNK

    def body(x_ref, o_ref):
        acc = None
        for k in range(nf):
            e_k = jnp.asarray(
                lax.broadcasted_iota(jnp.int32, (d, LANES), 0) + k * d
                == lax.broadcasted_iota(jnp.int32, (d, LANES), 1),
                dtype=jnp.bfloat16,
            )
            p = lax.dot_general(
                x_ref[k].astype(jnp.bfloat16),
                e_k,
                (((0,), (0,)), ((), ())),
                preferred_element_type=jnp.float32,
            )
            acc = p if acc is None else acc + p
        o_ref[0] = acc

    return pl.pallas_call(
        body,
        grid=(ngc, nv),
        in_specs=[
            pl.BlockSpec(
                (nf, d, VCHUNK), lambda gg, j: ((g0 * fold) // nf + gg, 0, j)
            )
        ],
        out_specs=pl.BlockSpec((1, VCHUNK, LANES), lambda gg, j: (gg, j, 0)),
        out_shape=jax.ShapeDtypeStruct((ngc, v, LANES), jnp.float32),
    )(t_dv)


def _sc_gather(idx2d, flat_tables, g, base_row, nw, d):
    f, b = idx2d.shape
    fold = LANES // d
    bw = b // nw
    nch = bw // CHUNK
    mesh = plsc.VectorSubcoreMesh(core_axis_name="c", subcore_axis_name="s")
    nc = 2

    @functools.partial(
        pl.kernel,
        mesh=mesh,
        compiler_params=pltpu.CompilerParams(use_tc_tiling_on_sc=False),
        out_type=jax.ShapeDtypeStruct((nw, bw, fold, d), jnp.float32),
    scratch_types=[
            pltpu.VMEM((bw,), jnp.int32),
            pltpu.VMEM((bw, d), jnp.float32),
            pltpu.SemaphoreType.DMA,
        ],
    )
    def k(idx_hbm, tab_hbm, out_hbm, idx_v, slab_v, sem):
        wid = lax.axis_index("s") * nc + lax.axis_index("c")

        for fp in range(fold):
            f_src = min(g * fold + fp, f - 1)
            pltpu.sync_copy(idx_hbm.at[f_src, pl.ds(wid * bw, bw)], idx_v)

            def tbody(c, carry, fp=fp):
                idx_v[pl.ds(c * 16, 16)] = (
                    idx_v[pl.ds(c * 16, 16)] * fold + (base_row + fp)
                )
                return carry

            lax.fori_loop(0, bw // 16, tbody, 0)

            handles = []
            for j in range(nch):
                h = pltpu.async_copy(
                    tab_hbm.at[idx_v.at[pl.ds(j * CHUNK, CHUNK)]],
                    slab_v.at[pl.ds(j * CHUNK, CHUNK)],
                    sem,
                )
                handles.append(h)
            for h in handles:
                h.wait()
            pltpu.sync_copy(slab_v, out_hbm.at[wid, :, fp])

    return k(idx2d, flat_tables)


def _tc_mlp(xs, w1s, b1, W2, b2, tb=1024):
    bsz, fd = xs[0].shape
    fold, d = 4, fd // 4
    h1 = w1s[0].shape[1]
    ed = W2.shape[1]
    ng = len(xs)

    def body(*refs):
        x_refs = refs[:ng]
        w_refs = refs[ng:2 * ng]
        b1_ref, w2_ref, b2_ref, o_ref = refs[2 * ng:]
        h = b1_ref[...]
        for xr, wr in zip(x_refs, w_refs):
            h = h + jnp.dot(xr[...], wr[...], preferred_element_type=jnp.float32)
        h = jnp.maximum(h, 0.0)
        o = jnp.dot(h, w2_ref[...], preferred_element_type=jnp.float32)
        o_ref[...] = jnp.maximum(o + b2_ref[...], 0.0)

    in_specs = (
        [pl.BlockSpec((tb, fold * d), lambda i: (i, 0)) for x in xs]
        + [pl.BlockSpec(w.shape, lambda i: (0, 0)) for w in w1s]
        + [
            pl.BlockSpec((1, h1), lambda i: (0, 0)),
            pl.BlockSpec(W2.shape, lambda i: (0, 0)),
            pl.BlockSpec((1, ed), lambda i: (0, 0)),
        ]
    )
    return pl.pallas_call(
        body,
        grid=(bsz // tb,),
        in_specs=in_specs,
        out_specs=pl.BlockSpec((tb, ed), lambda i: (i, 0)),
        out_shape=jax.ShapeDtypeStruct((bsz, ed), jnp.float32),
    )(*xs, *w1s, b1.reshape(1, h1), W2, b2.reshape(1, ed))


def kernel(indices, tables, W1, b1, W2, b2):
    f, b = indices.shape
    _, v, d = tables.shape
    nw = 32
    fold = LANES // d

    t_dv = jnp.transpose(tables, (0, 2, 1))
    idx32 = indices.astype(jnp.int32)

    ngrp = (f + fold - 1) // fold
    fpad = ngrp * fold
    w1_pad = jnp.pad(W1, ((0, (fpad - f) * d), (0, 0)))

    calls = []
    g = 0
    while g < ngrp:
        nf = min(fold, f - g * fold)
        ngc = 2 if (nf == fold and g + 1 < ngrp and f - (g + 1) * fold >= fold) else 1
        calls.append((g, ngc, nf))
        g += ngc
    calls.sort(key=lambda c: c[1])

    xs, w1s = [], []
    for g0, ngc, nf in calls:
        packed = _tc_repack(t_dv, g0, ngc, v, d, nf)
        flat_tab = packed.reshape(ngc * v * fold, d)
        for gg in range(ngc):
            g = g0 + gg
            rows = _sc_gather(idx32, flat_tab, g, gg * v * fold, nw, d)
            xs.append(rows.reshape(b, fold * d))
            w1s.append(w1_pad[g * fold * d : (g + 1) * fold * d])

    return _tc_mlp(xs, w1s, b1, W2, b2)


# speedup vs baseline: 2.5718x; 1.0114x over previous
import functools

import jax
import jax.numpy as jnp
from jax import lax
from jax.experimental import pallas as pl
from jax.experimental.pallas import tpu as pltpu
from jax.experimental.pallas import tpu_sc as plsc

CHUNK = 128
GROUP = 8
VCHUNK = 12800
LANES = 128


def _tc_repack(t_dv, g0, ngc, v, d, nf):
    fold = LANES // d
    nv = (v + VCHUNK - 1) // VCHUNK

    def body(x_ref, o_ref):
        acc = None
        for k in range(nf):
            e_k = jnp.asarray(
                lax.broadcasted_iota(jnp.int32, (d, LANES), 0) + k * d
                == lax.broadcasted_iota(jnp.int32, (d, LANES), 1),
                dtype=jnp.bfloat16,
            )
            p = lax.dot_general(
                x_ref[k].astype(jnp.bfloat16),
                e_k,
                (((0,), (0,)), ((), ())),
                preferred_element_type=jnp.float32,
            )
            acc = p if acc is None else acc + p
        o_ref[0] = acc

    return pl.pallas_call(
        body,
        grid=(ngc, nv),
        in_specs=[
            pl.BlockSpec(
                (nf, d, VCHUNK), lambda gg, j: ((g0 * fold) // nf + gg, 0, j)
            )
        ],
        out_specs=pl.BlockSpec((1, VCHUNK, LANES), lambda gg, j: (gg, j, 0)),
        out_shape=jax.ShapeDtypeStruct((ngc, v, LANES), jnp.float32),
    )(t_dv)


def _sc_gather(idx2d, flat_tables, g, base_row, nw, d):
    f, b = idx2d.shape
    fold = LANES // d
    bw = b // nw
    nch = bw // CHUNK
    mesh = plsc.VectorSubcoreMesh(core_axis_name="c", subcore_axis_name="s")
    nc = 2

    @functools.partial(
        pl.kernel,
        mesh=mesh,
        compiler_params=pltpu.CompilerParams(use_tc_tiling_on_sc=False),
        out_type=jax.ShapeDtypeStruct((nw, bw, fold, d), jnp.float32),
    scratch_types=[
            pltpu.VMEM((bw,), jnp.int32),
            pltpu.VMEM((bw, d), jnp.float32),
            pltpu.SemaphoreType.DMA,
        ],
    )
    def k(idx_hbm, tab_hbm, out_hbm, idx_v, slab_v, sem):
        wid = lax.axis_index("s") * nc + lax.axis_index("c")

        for fp in range(fold):
            f_src = min(g * fold + fp, f - 1)
            pltpu.sync_copy(idx_hbm.at[f_src, pl.ds(wid * bw, bw)], idx_v)

            def tbody(c, carry, fp=fp):
                idx_v[pl.ds(c * 16, 16)] = (
                    idx_v[pl.ds(c * 16, 16)] * fold + (base_row + fp)
                )
                return carry

            lax.fori_loop(0, bw // 16, tbody, 0)

            handles = []
            for j in range(nch):
                h = pltpu.async_copy(
                    tab_hbm.at[idx_v.at[pl.ds(j * CHUNK, CHUNK)]],
                    slab_v.at[pl.ds(j * CHUNK, CHUNK)],
                    sem,
                )
                handles.append(h)
            for h in handles:
                h.wait()
            pltpu.sync_copy(slab_v, out_hbm.at[wid, :, fp])

    return k(idx2d, flat_tables)


def _tc_mlp(xs, w1s, b1, W2, b2, tb=2048):
    bsz, fd = xs[0].shape
    fold, d = 4, fd // 4
    h1 = w1s[0].shape[1]
    ed = W2.shape[1]
    ng = len(xs)

    def body(*refs):
        x_refs = refs[:ng]
        w_refs = refs[ng:2 * ng]
        b1_ref, w2_ref, b2_ref, o_ref = refs[2 * ng:]
        h = b1_ref[...]
        for xr, wr in zip(x_refs, w_refs):
            h = h + jnp.dot(xr[...], wr[...], preferred_element_type=jnp.float32)
        h = jnp.maximum(h, 0.0)
        o = jnp.dot(h, w2_ref[...], preferred_element_type=jnp.float32)
        o_ref[...] = jnp.maximum(o + b2_ref[...], 0.0)

    in_specs = (
        [pl.BlockSpec((tb, fold * d), lambda i: (i, 0)) for x in xs]
        + [pl.BlockSpec(w.shape, lambda i: (0, 0)) for w in w1s]
        + [
            pl.BlockSpec((1, h1), lambda i: (0, 0)),
            pl.BlockSpec(W2.shape, lambda i: (0, 0)),
            pl.BlockSpec((1, ed), lambda i: (0, 0)),
        ]
    )
    return pl.pallas_call(
        body,
        grid=(bsz // tb,),
        in_specs=in_specs,
        out_specs=pl.BlockSpec((tb, ed), lambda i: (i, 0)),
        out_shape=jax.ShapeDtypeStruct((bsz, ed), jnp.float32),
    )(*xs, *w1s, b1.reshape(1, h1), W2, b2.reshape(1, ed))


def kernel(indices, tables, W1, b1, W2, b2):
    f, b = indices.shape
    _, v, d = tables.shape
    nw = 32
    fold = LANES // d

    t_dv = jnp.transpose(tables, (0, 2, 1))
    idx32 = indices.astype(jnp.int32)

    ngrp = (f + fold - 1) // fold
    fpad = ngrp * fold
    w1_pad = jnp.pad(W1, ((0, (fpad - f) * d), (0, 0)))

    calls = []
    g = 0
    while g < ngrp:
        nf = min(fold, f - g * fold)
        ngc = 2 if (nf == fold and g + 1 < ngrp and f - (g + 1) * fold >= fold) else 1
        calls.append((g, ngc, nf))
        g += ngc
    calls.sort(key=lambda c: c[1])

    xs, w1s = [], []
    for g0, ngc, nf in calls:
        packed = _tc_repack(t_dv, g0, ngc, v, d, nf)
        flat_tab = packed.reshape(ngc * v * fold, d)
        for gg in range(ngc):
            g = g0 + gg
            rows = _sc_gather(idx32, flat_tab, g, gg * v * fold, nw, d)
            xs.append(rows.reshape(b, fold * d))
            w1s.append(w1_pad[g * fold * d : (g + 1) * fold * d])

    return _tc_mlp(xs, w1s, b1, W2, b2)
